# Initial kernel scaffold; baseline (speedup 1.0000x reference)
import jax
import jax.numpy as jnp
from jax.experimental import pallas as pl


def kernel(prop_coords, prop_feats, orig_coords, orig_feats, W0, b0, g0, be0, W1, b1, g1, be1):
    raise NotImplementedError("write your pallas kernel here")


# speedup vs baseline: 16.1993x; 16.1993x over previous
import functools

import jax
import jax.numpy as jnp
from jax.experimental import pallas as pl

N_L = 4096
N_M = 16384
F1 = 64
F2 = 64
H = 128
TQ = 256
GRID = N_M // TQ
K = 3
EPS = 1e-5

_HIGH = jax.lax.Precision.HIGHEST


def _dot(a, b):
    return jax.lax.dot_general(a, b, (((1,), (0,)), ((), ())),
                               precision=_HIGH,
                               preferred_element_type=jnp.float32)


def _k1_body(q_ref, pT_ref, pf_ref, of_ref, w0c_ref, w0f_ref, w0i_ref,
             b0_ref, y0_ref, s_ref, ss_ref):
    q = q_ref[...]
    pT = pT_ref[...]
    qn = jnp.sum(q * q, axis=1, keepdims=True)
    pn = jnp.sum(pT * pT, axis=0, keepdims=True)
    g = _dot(q, pT)
    d2 = qn + pn - 2.0 * g

    iota = jax.lax.broadcasted_iota(jnp.int32, (TQ, N_L), 1)
    wmat = jnp.zeros((TQ, N_L), jnp.float32)
    wsum = jnp.zeros((TQ, 1), jnp.float32)
    for _ in range(K):
        m = jnp.min(d2, axis=1, keepdims=True)
        idx = jnp.min(jnp.where(d2 == m, iota, N_L), axis=1,
                      keepdims=True)
        onehot = (iota == idx)
        wk = 1.0 / jnp.maximum(m, 1e-12)
        wmat = wmat + jnp.where(onehot, wk, 0.0)
        wsum = wsum + wk
        d2 = jnp.where(onehot, jnp.inf, d2)
    wmat = wmat * (1.0 / wsum)

    interp = _dot(wmat, pf_ref[...])
    y0 = (_dot(q, w0c_ref[...]) + _dot(of_ref[...], w0f_ref[...])
          + _dot(interp, w0i_ref[...]) + b0_ref[...])
    y0_ref[...] = y0
    s_ref[...] = jnp.sum(y0, axis=0, keepdims=True)[None]
    ss_ref[...] = jnp.sum(y0 * y0, axis=0, keepdims=True)[None]


def _k2_body(y0_ref, sc_ref, sh_ref, w1_ref, b1_ref, y1_ref, s_ref, ss_ref):
    h = jnp.maximum(y0_ref[...] * sc_ref[...] + sh_ref[...], 0.0)
    y1 = _dot(h, w1_ref[...]) + b1_ref[...]
    y1_ref[...] = y1
    s_ref[...] = jnp.sum(y1, axis=0, keepdims=True)[None]
    ss_ref[...] = jnp.sum(y1 * y1, axis=0, keepdims=True)[None]


def _k3_body(y1_ref, sc_ref, sh_ref, o_ref):
    o_ref[...] = jnp.maximum(y1_ref[...] * sc_ref[...] + sh_ref[...], 0.0)


def _bn_coeffs(s, ss, g, be):
    mu = jnp.sum(s, axis=0)[0] / N_M
    var = jnp.sum(ss, axis=0)[0] / N_M - mu * mu
    scale = g / jnp.sqrt(var + EPS)
    shift = be - mu * scale
    return scale[None, :], shift[None, :]


@jax.jit
def kernel(prop_coords, prop_feats, orig_coords, orig_feats,
           W0, b0, g0, be0, W1, b1, g1, be1):
    qpad = jnp.pad(orig_coords, ((0, 0), (0, 5)))
    pT = jnp.pad(prop_coords, ((0, 0), (0, 5))).T
    w0c = jnp.pad(W0[:3], ((0, 5), (0, 0)))
    w0f = W0[3:3 + F2]
    w0i = W0[3 + F2:]

    full = lambda shp: pl.BlockSpec(shp, lambda i: (0,) * len(shp))
    row = lambda w: pl.BlockSpec((TQ, w), lambda i: (i, 0))
    stat = pl.BlockSpec((1, 1, H), lambda i: (i, 0, 0))
    statshape = jax.ShapeDtypeStruct((GRID, 1, H), jnp.float32)

    y0, s0, ss0 = pl.pallas_call(
        _k1_body,
        grid=(GRID,),
        in_specs=[row(8), full((8, N_L)), full((N_L, F1)), row(F2),
                  full((8, H)), full((F2, H)), full((F1, H)), full((1, H))],
        out_specs=[row(H), stat, stat],
        out_shape=[jax.ShapeDtypeStruct((N_M, H), jnp.float32),
                   statshape, statshape],
    )(qpad, pT, prop_feats, orig_feats, w0c, w0f, w0i, b0[None, :])

    sc0, sh0 = _bn_coeffs(s0, ss0, g0, be0)
    y1, s1, ss1 = pl.pallas_call(
        _k2_body,
        grid=(GRID,),
        in_specs=[row(H), full((1, H)), full((1, H)), full((H, H)),
                  full((1, H))],
        out_specs=[row(H), stat, stat],
        out_shape=[jax.ShapeDtypeStruct((N_M, H), jnp.float32),
                   statshape, statshape],
    )(y0, sc0, sh0, W1, b1[None, :])

    sc1, sh1 = _bn_coeffs(s1, ss1, g1, be1)
    out = pl.pallas_call(
        _k3_body,
        grid=(GRID,),
        in_specs=[row(H), full((1, H)), full((1, H))],
        out_specs=row(H),
        out_shape=jax.ShapeDtypeStruct((N_M, H), jnp.float32),
    )(y1, sc1, sh1)
    return out


# speedup vs baseline: 29.7460x; 1.8363x over previous
import functools

import jax
import jax.numpy as jnp
from jax.experimental import pallas as pl

N_L = 4096
N_M = 16384
F1 = 64
F2 = 64
H = 128
TQ = 256
GRID = N_M // TQ
K = 3
EPS = 1e-5

_HIGH = jax.lax.Precision.HIGHEST
_H3 = jax.lax.Precision.DEFAULT


def _dot(a, b, precision=_HIGH):
    return jax.lax.dot_general(a, b, (((1,), (0,)), ((), ())),
                               precision=precision,
                               preferred_element_type=jnp.float32)


def _k1_body(q_ref, pT_ref, pf_ref, of_ref, w0c_ref, w0f_ref, w0i_ref,
             b0_ref, y0_ref, s_ref, ss_ref):
    q = q_ref[...]
    pT = pT_ref[...]
    qn = jnp.sum(q * q, axis=1, keepdims=True)
    pn = jnp.sum(pT * pT, axis=0, keepdims=True)
    g = _dot(q, pT)
    d2 = qn + pn - 2.0 * g

    iota = jax.lax.broadcasted_iota(jnp.int32, (TQ, N_L), 1)
    wmat = jnp.zeros((TQ, N_L), jnp.float32)
    wsum = jnp.zeros((TQ, 1), jnp.float32)
    for _ in range(K):
        m = jnp.min(d2, axis=1, keepdims=True)
        idx = jnp.min(jnp.where(d2 == m, iota, N_L), axis=1,
                      keepdims=True)
        onehot = (iota == idx)
        wk = 1.0 / jnp.maximum(m, 1e-12)
        wmat = wmat + jnp.where(onehot, wk, 0.0)
        wsum = wsum + wk
        d2 = jnp.where(onehot, jnp.inf, d2)
    wmat = wmat * (1.0 / wsum)

    interp = _dot(wmat, pf_ref[...], _H3)
    y0 = (_dot(q, w0c_ref[...], _H3) + _dot(of_ref[...], w0f_ref[...], _H3)
          + _dot(interp, w0i_ref[...], _H3) + b0_ref[...])
    y0_ref[...] = y0
    s_ref[...] = jnp.sum(y0, axis=0, keepdims=True)[None]
    ss_ref[...] = jnp.sum(y0 * y0, axis=0, keepdims=True)[None]


def _k2_body(y0_ref, sc_ref, sh_ref, w1_ref, b1_ref, y1_ref, s_ref, ss_ref):
    h = jnp.maximum(y0_ref[...] * sc_ref[...] + sh_ref[...], 0.0)
    y1 = _dot(h, w1_ref[...], _H3) + b1_ref[...]
    y1_ref[...] = y1
    s_ref[...] = jnp.sum(y1, axis=0, keepdims=True)[None]
    ss_ref[...] = jnp.sum(y1 * y1, axis=0, keepdims=True)[None]


def _k3_body(y1_ref, sc_ref, sh_ref, o_ref):
    o_ref[...] = jnp.maximum(y1_ref[...] * sc_ref[...] + sh_ref[...], 0.0)


def _bn_coeffs(s, ss, g, be):
    mu = jnp.sum(s, axis=0)[0] / N_M
    var = jnp.sum(ss, axis=0)[0] / N_M - mu * mu
    scale = g / jnp.sqrt(var + EPS)
    shift = be - mu * scale
    return scale[None, :], shift[None, :]


@jax.jit
def kernel(prop_coords, prop_feats, orig_coords, orig_feats,
           W0, b0, g0, be0, W1, b1, g1, be1):
    qpad = jnp.pad(orig_coords, ((0, 0), (0, 5)))
    pT = jnp.pad(prop_coords, ((0, 0), (0, 5))).T
    w0c = jnp.pad(W0[:3], ((0, 5), (0, 0)))
    w0f = W0[3:3 + F2]
    w0i = W0[3 + F2:]

    full = lambda shp: pl.BlockSpec(shp, lambda i: (0,) * len(shp))
    row = lambda w: pl.BlockSpec((TQ, w), lambda i: (i, 0))
    stat = pl.BlockSpec((1, 1, H), lambda i: (i, 0, 0))
    statshape = jax.ShapeDtypeStruct((GRID, 1, H), jnp.float32)

    y0, s0, ss0 = pl.pallas_call(
        _k1_body,
        grid=(GRID,),
        in_specs=[row(8), full((8, N_L)), full((N_L, F1)), row(F2),
                  full((8, H)), full((F2, H)), full((F1, H)), full((1, H))],
        out_specs=[row(H), stat, stat],
        out_shape=[jax.ShapeDtypeStruct((N_M, H), jnp.float32),
                   statshape, statshape],
    )(qpad, pT, prop_feats, orig_feats, w0c, w0f, w0i, b0[None, :])

    sc0, sh0 = _bn_coeffs(s0, ss0, g0, be0)
    y1, s1, ss1 = pl.pallas_call(
        _k2_body,
        grid=(GRID,),
        in_specs=[row(H), full((1, H)), full((1, H)), full((H, H)),
                  full((1, H))],
        out_specs=[row(H), stat, stat],
        out_shape=[jax.ShapeDtypeStruct((N_M, H), jnp.float32),
                   statshape, statshape],
    )(y0, sc0, sh0, W1, b1[None, :])

    sc1, sh1 = _bn_coeffs(s1, ss1, g1, be1)
    out = pl.pallas_call(
        _k3_body,
        grid=(GRID,),
        in_specs=[row(H), full((1, H)), full((1, H))],
        out_specs=row(H),
        out_shape=jax.ShapeDtypeStruct((N_M, H), jnp.float32),
    )(y1, sc1, sh1)
    return out
